# baseline (speedup 1.0000x reference)
import functools

import jax
import jax.numpy as jnp
from jax import lax
from jax.experimental import pallas as pl
from jax.experimental.pallas import tpu as pltpu
from jax.experimental.pallas import tpu_sc as plsc

N = 32768
D_IN = 256
D_HID = 128
D_ATT = 64
B = 16
BLK = 2048
HALF = N // 2
NBLK_H = HALF // BLK

_NC = 2
_NS = 16
_TILES = _NC * _NS
_RPT = HALF // _TILES
_LANE = 16


def _enc_body(x_ref, W_enc_ref, b_enc_ref, V_ref, w_att_ref, W_bag_ref,
              g_ref):
    x = x_ref[...].astype(jnp.bfloat16)
    h = jnp.dot(x, W_enc_ref[...].astype(jnp.bfloat16),
                preferred_element_type=jnp.float32)
    h = h + b_enc_ref[...]
    t = jnp.tanh(jnp.dot(h.astype(jnp.bfloat16),
                         V_ref[...].astype(jnp.bfloat16),
                         preferred_element_type=jnp.float32))
    a_row = lax.dot_general(w_att_ref[...], t, (((0,), (1,)), ((), ())),
                            preferred_element_type=jnp.float32)
    e_row = jnp.exp(a_row)
    p_row = lax.dot_general(W_bag_ref[...], h, (((0,), (1,)), ((), ())),
                            preferred_element_type=jnp.float32)
    g_ref[...] = jnp.concatenate([e_row * p_row, e_row], axis=0)


def _encode_half(x, W_enc, b_enc, V, w_att, W_bag, half):
    blk_off = half * NBLK_H
    return pl.pallas_call(
        _enc_body,
        grid=(NBLK_H,),
        in_specs=[
            pl.BlockSpec((BLK, D_IN), lambda i: (i + blk_off, 0)),
            pl.BlockSpec((D_IN, D_HID), lambda i: (0, 0)),
            pl.BlockSpec((1, D_HID), lambda i: (0, 0)),
            pl.BlockSpec((D_HID, D_ATT), lambda i: (0, 0)),
            pl.BlockSpec((D_ATT, 1), lambda i: (0, 0)),
            pl.BlockSpec((D_HID, 1), lambda i: (0, 0)),
        ],
        out_specs=pl.BlockSpec((2, BLK), lambda i: (0, i)),
        out_shape=jax.ShapeDtypeStruct((2, HALF), jnp.float32),
    )(x, W_enc, b_enc.reshape(1, D_HID), V, w_att, W_bag)


_SC_MESH = plsc.VectorSubcoreMesh(core_axis_name="c", subcore_axis_name="s",
                                  num_cores=_NC, num_subcores=_NS)


def _make_sc(half):
    off = half * HALF

    @functools.partial(
        pl.kernel,
        out_type=jax.ShapeDtypeStruct((_TILES, 2, B, _LANE), jnp.float32),
        mesh=_SC_MESH,
        scratch_types=[
            pltpu.VMEM((_RPT,), jnp.float32),
            pltpu.VMEM((_RPT,), jnp.float32),
            pltpu.VMEM((B, _LANE), jnp.float32),
            pltpu.VMEM((B, _LANE), jnp.float32),
            pltpu.VMEM((2 * B,), jnp.int32),
            pltpu.SemaphoreType.DMA,
            pltpu.SemaphoreType.DMA,
            pltpu.SemaphoreType.DMA,
        ],
    )
    def _sc_bag_sums(g_hbm, bs_hbm, out_hbm, wbuf, ebuf, acc_w, acc_e, bs_v,
                     s0, s1, s2):
        cid = lax.axis_index("c")
        sid = lax.axis_index("s")
        wid = sid * _NC + cid
        lo = wid * _RPT
        glo = off + lo

        d0 = pltpu.async_copy(bs_hbm, bs_v, s0)
        d1 = pltpu.async_copy(g_hbm.at[0, pl.ds(lo, _RPT)], wbuf, s1)
        d2 = pltpu.async_copy(g_hbm.at[1, pl.ds(lo, _RPT)], ebuf, s2)

        lane = lax.iota(jnp.int32, _LANE)
        zero = jnp.zeros((_LANE,), jnp.float32)
        for j in range(B):
            acc_w[j] = zero
            acc_e[j] = zero

        d0.wait()
        d1.wait()
        d2.wait()
        bs_lo = bs_v[pl.ds(0, B)]
        bs_hi = bs_v[pl.ds(B, B)]

        for j in range(B):
            r0 = jnp.maximum(glo, bs_lo[j])
            r1 = jnp.minimum(glo + _RPT, bs_hi[j])

            @pl.when(r1 > r0)
            def _(j=j, r0=r0, r1=r1):
                v0 = (r0 - glo) // _LANE
                v1 = (r1 - glo + _LANE - 1) // _LANE

                def body(v, accs):
                    aw, ae = accs
                    base = v * _LANE
                    idx = glo + base + lane
                    m = (idx >= r0) & (idx < r1)
                    wv = wbuf[pl.ds(base, _LANE)]
                    ev = ebuf[pl.ds(base, _LANE)]
                    return (aw + jnp.where(m, wv, 0.0),
                            ae + jnp.where(m, ev, 0.0))

                aw, ae = lax.fori_loop(v0, v1, body, (zero, zero))
                acc_w[j] = aw
                acc_e[j] = ae

        pltpu.sync_copy(acc_w, out_hbm.at[wid, 0])
        pltpu.sync_copy(acc_e, out_hbm.at[wid, 1])

    return _sc_bag_sums


_SC_HALF0 = _make_sc(0)
_SC_HALF1 = _make_sc(1)


def _fin_body(zpa_ref, zpb_ref, b_bag_ref, out_ref):
    zp = zpa_ref[...] + zpb_ref[...]
    num = jnp.sum(zp[:, 0, :, :], axis=(0, 2))
    den = jnp.sum(zp[:, 1, :, :], axis=(0, 2))
    yhat = num / jnp.where(den > 0, den, 1.0) + b_bag_ref[0, 0]
    out_ref[...] = yhat.reshape(B, 1)


def _finalize(zpa, zpb, b_bag):
    return pl.pallas_call(
        _fin_body,
        out_shape=jax.ShapeDtypeStruct((B, 1), jnp.float32),
    )(zpa, zpb, b_bag.reshape(1, 1))


def kernel(x, bag_sizes, W_enc, b_enc, V, w_att, W_ins, b_ins, W_bag, b_bag):
    bs_pad = jnp.concatenate([bag_sizes[:B], bag_sizes[1:]])
    ga = _encode_half(x, W_enc, b_enc, V, w_att, W_bag, 0)
    zpa = _SC_HALF0(ga, bs_pad)
    gb = _encode_half(x, W_enc, b_enc, V, w_att, W_bag, 1)
    zpb = _SC_HALF1(gb, bs_pad)
    return _finalize(zpa, zpb, b_bag)


# speedup vs baseline: 1.0446x; 1.0446x over previous
import functools

import jax
import jax.numpy as jnp
from jax import lax
from jax.experimental import pallas as pl
from jax.experimental.pallas import tpu as pltpu
from jax.experimental.pallas import tpu_sc as plsc

N = 32768
D_IN = 256
D_HALF = D_IN // 2
D_HID = 128
D_ATT = 64
B = 16
BLK = 2048
NBLK = N // BLK

_NC = 2
_NS = 16
_TILES = _NC * _NS
_RPT = N // _TILES
_LANE = 16


def _enc_body(x1_ref, x2_ref, W_enc_ref, b_enc_ref, V_ref, w_att_ref,
              W_bag_ref, g_ref):
    W = W_enc_ref[...].astype(jnp.bfloat16)
    h = jnp.dot(x1_ref[...].astype(jnp.bfloat16), W[:D_HALF],
                preferred_element_type=jnp.float32)
    h = h + jnp.dot(x2_ref[...].astype(jnp.bfloat16), W[D_HALF:],
                    preferred_element_type=jnp.float32)
    h = h + b_enc_ref[...]
    t = jnp.tanh(jnp.dot(h.astype(jnp.bfloat16),
                         V_ref[...].astype(jnp.bfloat16),
                         preferred_element_type=jnp.float32))
    a_row = lax.dot_general(w_att_ref[...], t, (((0,), (1,)), ((), ())),
                            preferred_element_type=jnp.float32)
    e_row = jnp.exp(a_row)
    p_row = lax.dot_general(W_bag_ref[...], h, (((0,), (1,)), ((), ())),
                            preferred_element_type=jnp.float32)
    g_ref[...] = jnp.concatenate([e_row * p_row, e_row], axis=0)


def _encode(x, W_enc, b_enc, V, w_att, W_bag):
    return pl.pallas_call(
        _enc_body,
        grid=(NBLK,),
        in_specs=[
            pl.BlockSpec((BLK, D_HALF), lambda i: (i, 0)),
            pl.BlockSpec((BLK, D_HALF), lambda i: (i, 1)),
            pl.BlockSpec((D_IN, D_HID), lambda i: (0, 0)),
            pl.BlockSpec((1, D_HID), lambda i: (0, 0)),
            pl.BlockSpec((D_HID, D_ATT), lambda i: (0, 0)),
            pl.BlockSpec((D_ATT, 1), lambda i: (0, 0)),
            pl.BlockSpec((D_HID, 1), lambda i: (0, 0)),
        ],
        out_specs=pl.BlockSpec((2, BLK), lambda i: (0, i)),
        out_shape=jax.ShapeDtypeStruct((2, N), jnp.float32),
    )(x, x, W_enc, b_enc.reshape(1, D_HID), V, w_att, W_bag)


_SC_MESH = plsc.VectorSubcoreMesh(core_axis_name="c", subcore_axis_name="s",
                                  num_cores=_NC, num_subcores=_NS)


@functools.partial(
    pl.kernel,
    out_type=jax.ShapeDtypeStruct((_TILES, 2, B, _LANE), jnp.float32),
    mesh=_SC_MESH,
    scratch_types=[
        pltpu.VMEM((_RPT,), jnp.float32),
        pltpu.VMEM((_RPT,), jnp.float32),
        pltpu.VMEM((B, _LANE), jnp.float32),
        pltpu.VMEM((B, _LANE), jnp.float32),
        pltpu.VMEM((2 * B,), jnp.int32),
        pltpu.SemaphoreType.DMA,
        pltpu.SemaphoreType.DMA,
        pltpu.SemaphoreType.DMA,
    ],
)
def _sc_bag_sums(g_hbm, bs_hbm, out_hbm, wbuf, ebuf, acc_w, acc_e, bs_v,
                 s0, s1, s2):
    cid = lax.axis_index("c")
    sid = lax.axis_index("s")
    wid = sid * _NC + cid
    lo = wid * _RPT

    d0 = pltpu.async_copy(bs_hbm, bs_v, s0)
    d1 = pltpu.async_copy(g_hbm.at[0, pl.ds(lo, _RPT)], wbuf, s1)
    d2 = pltpu.async_copy(g_hbm.at[1, pl.ds(lo, _RPT)], ebuf, s2)

    lane = lax.iota(jnp.int32, _LANE)
    zero = jnp.zeros((_LANE,), jnp.float32)
    for j in range(B):
        acc_w[j] = zero
        acc_e[j] = zero

    d0.wait()
    d1.wait()
    d2.wait()
    bs_lo = bs_v[pl.ds(0, B)]
    bs_hi = bs_v[pl.ds(B, B)]

    for j in range(B):
        r0 = jnp.maximum(lo, bs_lo[j])
        r1 = jnp.minimum(lo + _RPT, bs_hi[j])

        @pl.when(r1 > r0)
        def _(j=j, r0=r0, r1=r1):
            v0 = (r0 - lo) // _LANE
            v1 = (r1 - lo + _LANE - 1) // _LANE

            def body(v, accs):
                aw, ae = accs
                base = v * _LANE
                idx = lo + base + lane
                m = (idx >= r0) & (idx < r1)
                wv = wbuf[pl.ds(base, _LANE)]
                ev = ebuf[pl.ds(base, _LANE)]
                return (aw + jnp.where(m, wv, 0.0),
                        ae + jnp.where(m, ev, 0.0))

            aw, ae = lax.fori_loop(v0, v1, body, (zero, zero))
            acc_w[j] = aw
            acc_e[j] = ae

    pltpu.sync_copy(acc_w, out_hbm.at[wid, 0])
    pltpu.sync_copy(acc_e, out_hbm.at[wid, 1])


def _fin_body(zp_ref, b_bag_ref, out_ref):
    zp = zp_ref[...]
    num = jnp.sum(zp[:, 0, :, :], axis=(0, 2))
    den = jnp.sum(zp[:, 1, :, :], axis=(0, 2))
    yhat = num / jnp.where(den > 0, den, 1.0) + b_bag_ref[0, 0]
    out_ref[...] = yhat.reshape(B, 1)


def _finalize(zp, b_bag):
    return pl.pallas_call(
        _fin_body,
        out_shape=jax.ShapeDtypeStruct((B, 1), jnp.float32),
    )(zp, b_bag.reshape(1, 1))


def kernel(x, bag_sizes, W_enc, b_enc, V, w_att, W_ins, b_ins, W_bag, b_bag):
    bs_pad = jnp.concatenate([bag_sizes[:B], bag_sizes[1:]])
    g = _encode(x, W_enc, b_enc, V, w_att, W_bag)
    zp = _sc_bag_sums(g, bs_pad)
    return _finalize(zp, b_bag)


# speedup vs baseline: 1.0812x; 1.0350x over previous
import functools

import jax
import jax.numpy as jnp
from jax import lax
from jax.experimental import pallas as pl
from jax.experimental.pallas import tpu as pltpu
from jax.experimental.pallas import tpu_sc as plsc

N = 32768
D_IN = 256
D_HALF = D_IN // 2
D_HID = 128
D_ATT = 64
B = 16
BLK = 2048
NBLK = N // BLK

_NC = 2
_NS = 16
_TILES = _NC * _NS
_RPT = N // _TILES
_LANE = 16


def _enc_body(x1_ref, x2_ref, x3_ref, x4_ref, W_enc_ref, b_enc_ref, V_ref,
              w_att_ref, W_bag_ref, g_ref):
    W = W_enc_ref[...].astype(jnp.bfloat16)
    Wa, Wb = W[:D_HALF], W[D_HALF:]
    h_top = jnp.dot(x1_ref[...].astype(jnp.bfloat16), Wa,
                    preferred_element_type=jnp.float32)
    h_top = h_top + jnp.dot(x2_ref[...].astype(jnp.bfloat16), Wb,
                            preferred_element_type=jnp.float32)
    h_bot = jnp.dot(x3_ref[...].astype(jnp.bfloat16), Wa,
                    preferred_element_type=jnp.float32)
    h_bot = h_bot + jnp.dot(x4_ref[...].astype(jnp.bfloat16), Wb,
                            preferred_element_type=jnp.float32)
    h = jnp.concatenate([h_top, h_bot], axis=0)
    h = h + b_enc_ref[...]
    t = jnp.tanh(jnp.dot(h.astype(jnp.bfloat16),
                         V_ref[...].astype(jnp.bfloat16),
                         preferred_element_type=jnp.float32))
    a_row = lax.dot_general(w_att_ref[...], t, (((0,), (1,)), ((), ())),
                            preferred_element_type=jnp.float32)
    e_row = jnp.exp(a_row)
    p_row = lax.dot_general(W_bag_ref[...], h, (((0,), (1,)), ((), ())),
                            preferred_element_type=jnp.float32)
    g_ref[...] = jnp.concatenate([e_row * p_row, e_row], axis=0)


def _encode(x, W_enc, b_enc, V, w_att, W_bag):
    return pl.pallas_call(
        _enc_body,
        grid=(NBLK,),
        in_specs=[
            pl.BlockSpec((BLK // 2, D_HALF), lambda i: (2 * i, 0)),
            pl.BlockSpec((BLK // 2, D_HALF), lambda i: (2 * i, 1)),
            pl.BlockSpec((BLK // 2, D_HALF), lambda i: (2 * i + 1, 0)),
            pl.BlockSpec((BLK // 2, D_HALF), lambda i: (2 * i + 1, 1)),
            pl.BlockSpec((D_IN, D_HID), lambda i: (0, 0)),
            pl.BlockSpec((1, D_HID), lambda i: (0, 0)),
            pl.BlockSpec((D_HID, D_ATT), lambda i: (0, 0)),
            pl.BlockSpec((D_ATT, 1), lambda i: (0, 0)),
            pl.BlockSpec((D_HID, 1), lambda i: (0, 0)),
        ],
        out_specs=pl.BlockSpec((2, BLK), lambda i: (0, i)),
        out_shape=jax.ShapeDtypeStruct((2, N), jnp.float32),
    )(x, x, x, x, W_enc, b_enc.reshape(1, D_HID), V, w_att, W_bag)


_SC_MESH = plsc.VectorSubcoreMesh(core_axis_name="c", subcore_axis_name="s",
                                  num_cores=_NC, num_subcores=_NS)


@functools.partial(
    pl.kernel,
    out_type=jax.ShapeDtypeStruct((_TILES, 2, B, _LANE), jnp.float32),
    mesh=_SC_MESH,
    scratch_types=[
        pltpu.VMEM((_RPT,), jnp.float32),
        pltpu.VMEM((_RPT,), jnp.float32),
        pltpu.VMEM((B, _LANE), jnp.float32),
        pltpu.VMEM((B, _LANE), jnp.float32),
        pltpu.VMEM((2 * B,), jnp.int32),
        pltpu.SemaphoreType.DMA,
        pltpu.SemaphoreType.DMA,
        pltpu.SemaphoreType.DMA,
    ],
)
def _sc_bag_sums(g_hbm, bs_hbm, out_hbm, wbuf, ebuf, acc_w, acc_e, bs_v,
                 s0, s1, s2):
    cid = lax.axis_index("c")
    sid = lax.axis_index("s")
    wid = sid * _NC + cid
    lo = wid * _RPT

    d0 = pltpu.async_copy(bs_hbm, bs_v, s0)
    d1 = pltpu.async_copy(g_hbm.at[0, pl.ds(lo, _RPT)], wbuf, s1)
    d2 = pltpu.async_copy(g_hbm.at[1, pl.ds(lo, _RPT)], ebuf, s2)

    lane = lax.iota(jnp.int32, _LANE)
    zero = jnp.zeros((_LANE,), jnp.float32)
    for j in range(B):
        acc_w[j] = zero
        acc_e[j] = zero

    d0.wait()
    d1.wait()
    d2.wait()
    bs_lo = bs_v[pl.ds(0, B)]
    bs_hi = bs_v[pl.ds(B, B)]

    for j in range(B):
        r0 = jnp.maximum(lo, bs_lo[j])
        r1 = jnp.minimum(lo + _RPT, bs_hi[j])

        @pl.when(r1 > r0)
        def _(j=j, r0=r0, r1=r1):
            v0 = (r0 - lo) // _LANE
            v1 = (r1 - lo + _LANE - 1) // _LANE

            def body(v, accs):
                aw, ae = accs
                base = v * _LANE
                idx = lo + base + lane
                m = (idx >= r0) & (idx < r1)
                wv = wbuf[pl.ds(base, _LANE)]
                ev = ebuf[pl.ds(base, _LANE)]
                return (aw + jnp.where(m, wv, 0.0),
                        ae + jnp.where(m, ev, 0.0))

            aw, ae = lax.fori_loop(v0, v1, body, (zero, zero))
            acc_w[j] = aw
            acc_e[j] = ae

    pltpu.sync_copy(acc_w, out_hbm.at[wid, 0])
    pltpu.sync_copy(acc_e, out_hbm.at[wid, 1])


def _fin_body(zp_ref, b_bag_ref, out_ref):
    zp = zp_ref[...]
    num = jnp.sum(zp[:, 0, :, :], axis=(0, 2))
    den = jnp.sum(zp[:, 1, :, :], axis=(0, 2))
    yhat = num / jnp.where(den > 0, den, 1.0) + b_bag_ref[0, 0]
    out_ref[...] = yhat.reshape(B, 1)


def _finalize(zp, b_bag):
    return pl.pallas_call(
        _fin_body,
        out_shape=jax.ShapeDtypeStruct((B, 1), jnp.float32),
    )(zp, b_bag.reshape(1, 1))


def kernel(x, bag_sizes, W_enc, b_enc, V, w_att, W_ins, b_ins, W_bag, b_bag):
    bs_pad = jnp.concatenate([bag_sizes[:B], bag_sizes[1:]])
    g = _encode(x, W_enc, b_enc, V, w_att, W_bag)
    zp = _sc_bag_sums(g, bs_pad)
    return _finalize(zp, b_bag)
